# baseline (speedup 1.0000x reference)
import functools

import jax
import jax.numpy as jnp
from jax import lax
from jax.experimental import pallas as pl
from jax.experimental.pallas import tpu as pltpu
from jax.experimental.pallas import tpu_sc as plsc

NUM_GROUPS = 64
MEMBERS = 8
D = 64
B = 4096
H = 96

_TP_LANES = 4096


def _tp_body(in_ref, out_ref):
    ident = (lax.broadcasted_iota(jnp.int32, (D, D), 0) ==
             lax.broadcasted_iota(jnp.int32, (D, D), 1)).astype(jnp.float32)
    t = lax.dot_general(in_ref[...], ident, (((0,), (0,)), ((), ())),
                        preferred_element_type=jnp.float32)
    half = _TP_LANES // 2
    out_ref[...] = jnp.concatenate([t[0:half, :], t[half:_TP_LANES, :]], axis=1)


def _pair_table(tableT):
    V = tableT.shape[1]
    nblk = (V + _TP_LANES - 1) // _TP_LANES
    n_rows = (_TP_LANES // 2) * nblk
    return pl.pallas_call(
        _tp_body,
        grid=(nblk,),
        in_specs=[pl.BlockSpec((D, _TP_LANES), lambda i: (0, i))],
        out_specs=pl.BlockSpec((_TP_LANES // 2, 2 * D), lambda i: (i, 0)),
        out_shape=jax.ShapeDtypeStruct((n_rows, 2 * D), jnp.float32),
    )(tableT)


@functools.cache
def _sc_gather(V2, Bb):
    info = plsc.get_sparse_core_info()
    NC, NS = info.num_cores, info.num_subcores
    NW = NC * NS
    b_per_w = Bb // NW
    mesh = plsc.VectorSubcoreMesh(core_axis_name="c", subcore_axis_name="s")

    @functools.partial(
        pl.kernel,
        mesh=mesh,
        out_type=jax.ShapeDtypeStruct((Bb, 2 * D), jnp.float32),
        scratch_types=[
            pltpu.VMEM((b_per_w,), jnp.int32),
            pltpu.VMEM((b_per_w, 2 * D), jnp.float32),
            pltpu.SemaphoreType.DMA,
        ],
    )
    def gather(table_hbm, idx_hbm, out_hbm, idx_v, rows_v, sem):
        wid = lax.axis_index("s") * NC + lax.axis_index("c")
        base = wid * b_per_w
        pltpu.sync_copy(idx_hbm.at[pl.ds(base, b_per_w)], idx_v)
        pltpu.async_copy(table_hbm.at[idx_v], rows_v, sem).wait()
        pltpu.sync_copy(rows_v, out_hbm.at[pl.ds(base, b_per_w)])

    return gather


def _tc_body(user_ref, group_ref, item_ref, parity_ref, W1_ref, b1_ref,
             W2_ref, b2_ref, W3_ref, b3_ref, Wp1_ref, bp1_ref, wp2_ref,
             bp2_ref, out_ref):
    u_iota = lax.broadcasted_iota(jnp.int32, (NUM_GROUPS, NUM_GROUPS * MEMBERS), 1)
    g_iota = lax.broadcasted_iota(jnp.int32, (NUM_GROUPS, NUM_GROUPS * MEMBERS), 0)
    pool = jnp.where(u_iota // MEMBERS == g_iota, 1.0 / MEMBERS, 0.0)
    ua = jnp.maximum(jnp.dot(pool, user_ref[...],
                             preferred_element_type=jnp.float32), 0.0)
    h = jnp.maximum(jnp.dot(ua, W1_ref[...],
                            preferred_element_type=jnp.float32) + b1_ref[...], 0.0)
    h = jnp.maximum(jnp.dot(h, W2_ref[...],
                            preferred_element_type=jnp.float32) + b2_ref[...], 0.0)
    zmu = jnp.dot(h, W3_ref[...],
                  preferred_element_type=jnp.float32) + b3_ref[...]
    onehot = (group_ref[...] ==
              lax.broadcasted_iota(jnp.int32, (B, NUM_GROUPS), 1)
              ).astype(jnp.float32)
    Z = jnp.dot(onehot, zmu, preferred_element_type=jnp.float32)
    E = jnp.where(parity_ref[...] == 0, item_ref[:, 0:D], item_ref[:, D:2 * D])
    A = Wp1_ref[0:D, :]
    Bm = Wp1_ref[D:2 * D, :]
    C = Wp1_ref[2 * D:3 * D, :]
    h2 = (jnp.dot(Z * E, A, preferred_element_type=jnp.float32)
          + jnp.dot(Z, Bm, preferred_element_type=jnp.float32)
          + jnp.dot(E, C, preferred_element_type=jnp.float32)
          + bp1_ref[...])
    h2 = jnp.maximum(h2, 0.0)
    y = jnp.sum(h2 * wp2_ref[...], axis=1, keepdims=True) + bp2_ref[...]
    out_ref[...] = jax.nn.sigmoid(y)


@jax.jit
def _tc_call(user_slice, group2d, item_pairs, parity2d, W1, b1, W2, b2,
             W3z, b3z, Wp1, bp1, wp2row, bp2):
    return pl.pallas_call(
        _tc_body,
        out_shape=jax.ShapeDtypeStruct((B, 1), jnp.float32),
    )(user_slice, group2d, item_pairs, parity2d, W1, b1, W2, b2, W3z, b3z,
      Wp1, bp1, wp2row, bp2)


def kernel(group_inputs, item_inputs, user_table, item_table,
           W1, b1, W2, b2, W3, b3, Wp1, bp1, Wp2, bp2):
    items = item_inputs.astype(jnp.int32)
    table2 = _pair_table(item_table.T)
    half = _TP_LANES // 2
    pair_idx = (items // _TP_LANES) * half + items % half
    parity2d = ((items // half) & 1).reshape(B, 1)
    item_pairs = _sc_gather(table2.shape[0], B)(table2, pair_idx)
    user_slice = user_table[:NUM_GROUPS * MEMBERS]
    group2d = group_inputs.astype(jnp.int32).reshape(B, 1)
    return _tc_call(
        user_slice, group2d, item_pairs, parity2d,
        W1, b1.reshape(1, H), W2, b2.reshape(1, H),
        W3[:, :D], b3[:D].reshape(1, D),
        Wp1, bp1.reshape(1, 8), Wp2.reshape(1, 8), bp2.reshape(1, 1))
